# baseline (speedup 1.0000x reference)
import functools

import jax
import jax.numpy as jnp
from jax import lax
from jax.experimental import pallas as pl
from jax.experimental.pallas import tpu as pltpu
from jax.experimental.pallas import tpu_sc as plsc

N = 10000
E = 320000
D = 128
H = 128
C = 40

NC = 2
NS = 16
NW = NC * NS
RPT = 632
N_PAD = RPT * NS
CHUNK = 80
NCHUNK = 126
EPT = CHUNK * NCHUNK
E_PAD = NW * EPT

BN = 1000
GRID = N // BN


def _make_sc_agg(with_deg: bool):
  mesh = plsc.VectorSubcoreMesh(core_axis_name="c", subcore_axis_name="s")
  out_type = [jax.ShapeDtypeStruct((NC, N_PAD, D), jnp.float32)]
  scratch = [
      pltpu.VMEM((CHUNK,), jnp.int32),
      pltpu.VMEM((CHUNK,), jnp.int32),
      pltpu.VMEM((CHUNK,), jnp.int32),
      pltpu.VMEM((CHUNK,), jnp.int32),
      pltpu.VMEM((CHUNK, D), jnp.float32),
      pltpu.VMEM((CHUNK, D), jnp.float32),
      pltpu.VMEM_SHARED((N_PAD, D), jnp.float32),
      pltpu.SemaphoreType.DMA,
      pltpu.SemaphoreType.DMA,
  ]
  if with_deg:
    out_type.append(jax.ShapeDtypeStruct((NC, N_PAD, D), jnp.float32))
    scratch.append(pltpu.VMEM((CHUNK, D), jnp.float32))

  def body(*refs):
    if with_deg:
      (h_hbm, src_hbm, dst_hbm, z_d, ones_hbm, out_agg, out_deg,
       src_v, dst_v, src_w, dst_w, rows_v, rows_w, agg_sh, sem, sem2,
       ones_v) = refs
    else:
      (h_hbm, src_hbm, dst_hbm, z_d, out_agg,
       src_v, dst_v, src_w, dst_w, rows_v, rows_w, agg_sh, sem, sem2) = refs
    cid = lax.axis_index("c")
    sid = lax.axis_index("s")
    base = (cid * NS + sid) * EPT
    my_rows = pl.ds(sid * RPT, RPT)

    pltpu.sync_copy(z_d, agg_sh.at[my_rows])
    if with_deg:
      pltpu.sync_copy(ones_hbm, ones_v)
    plsc.subcore_barrier()

    if with_deg:
      def dstep(i, carry):
        off = base + i * CHUNK
        pltpu.sync_copy(dst_hbm.at[pl.ds(off, CHUNK)], dst_v)
        pltpu.sync_copy(ones_v, agg_sh.at[dst_v], add=True)
        return carry

      lax.fori_loop(0, NCHUNK, dstep, 0)
      plsc.subcore_barrier()
      pltpu.sync_copy(agg_sh.at[my_rows], out_deg.at[cid, my_rows])
      pltpu.sync_copy(z_d, agg_sh.at[my_rows])
      plsc.subcore_barrier()

    def step(i, carry):
      off = base + i * CHUNK
      pltpu.sync_copy(src_hbm.at[pl.ds(off, CHUNK)], src_v)
      pltpu.sync_copy(dst_hbm.at[pl.ds(off, CHUNK)], dst_v)
      pltpu.async_copy(h_hbm.at[src_v], rows_v, sem).wait()
      pltpu.sync_copy(rows_v, agg_sh.at[dst_v], add=True)
      return carry

    lax.fori_loop(0, NCHUNK, step, 0)
    plsc.subcore_barrier()
    pltpu.sync_copy(agg_sh.at[my_rows], out_agg.at[cid, my_rows])

  return pl.kernel(body, out_type=out_type, mesh=mesh, scratch_types=scratch)


_sc_agg_deg = _make_sc_agg(True)
_sc_agg = _make_sc_agg(False)


def _tc_layer_body(h_ref, aggA_ref, aggB_ref, degA_ref, degB_ref,
                   ws_ref, wn_ref, b_ref, out_ref):
  deg = degA_ref[:, :1] + degB_ref[:, :1]
  inv = 1.0 / jnp.maximum(deg, 1.0)
  agg = (aggA_ref[...] + aggB_ref[...]) * inv
  acc = jnp.dot(h_ref[...], ws_ref[...], preferred_element_type=jnp.float32)
  acc += jnp.dot(agg, wn_ref[...], preferred_element_type=jnp.float32)
  out_ref[...] = jnp.maximum(acc + b_ref[...], 0.0)


def _tc_final_body(h_ref, aggA_ref, aggB_ref, degA_ref, degB_ref,
                   ws_ref, wn_ref, b_ref, wp_ref, bp_ref,
                   score_ref, pls_ref):
  i = pl.program_id(0)
  deg = degA_ref[:, :1] + degB_ref[:, :1]
  inv = 1.0 / jnp.maximum(deg, 1.0)
  agg = (aggA_ref[...] + aggB_ref[...]) * inv
  acc = jnp.dot(h_ref[...], ws_ref[...], preferred_element_type=jnp.float32)
  acc += jnp.dot(agg, wn_ref[...], preferred_element_type=jnp.float32)
  h2 = jnp.maximum(acc + b_ref[...], 0.0)

  @pl.when(i == 0)
  def _():
    pls_ref[...] = jnp.zeros_like(pls_ref)

  pls_ref[...] += jnp.sum(h2, axis=0, keepdims=True)

  @pl.when(i == pl.num_programs(0) - 1)
  def _():
    pls = pls_ref[...] * (1.0 / N)
    pls_ref[...] = pls
    score_ref[...] = (
        jnp.dot(pls, wp_ref[...], preferred_element_type=jnp.float32)
        + bp_ref[...])


def _row_blocks(n_cols):
  return pl.BlockSpec((BN, n_cols), lambda i: (i, 0))


def _full(shape):
  return pl.BlockSpec(shape, lambda i: tuple(0 for _ in shape))


_tc_layer = pl.pallas_call(
    _tc_layer_body,
    out_shape=jax.ShapeDtypeStruct((N, H), jnp.float32),
    grid=(GRID,),
    in_specs=[
        _row_blocks(D), _row_blocks(D), _row_blocks(D),
        _row_blocks(D), _row_blocks(D),
        _full((D, H)), _full((D, H)), _full((1, H)),
    ],
    out_specs=_row_blocks(H),
)

_tc_final = pl.pallas_call(
    _tc_final_body,
    out_shape=(
        jax.ShapeDtypeStruct((1, C), jnp.float32),
        jax.ShapeDtypeStruct((1, H), jnp.float32),
    ),
    grid=(GRID,),
    in_specs=[
        _row_blocks(H), _row_blocks(H), _row_blocks(H),
        _row_blocks(D), _row_blocks(D),
        _full((H, H)), _full((H, H)), _full((1, H)),
        _full((H, C)), _full((1, C)),
    ],
    out_specs=(_full((1, C)), _full((1, H))),
)


def kernel(inputs, edge_index, W_self0, W_neigh0, b0, W_self1, W_neigh1, b1,
           W_pred, b_pred):
  pad = E_PAD - E
  src = jnp.concatenate([edge_index[0], jnp.zeros((pad,), jnp.int32)])
  dst = jnp.concatenate([edge_index[1], jnp.full((pad,), N_PAD - 1, jnp.int32)])
  z_d = jnp.zeros((RPT, D), jnp.float32)
  ones = jnp.ones((CHUNK, D), jnp.float32)

  agg0, deg = _sc_agg_deg(inputs, src, dst, z_d, ones)
  h1 = _tc_layer(inputs, agg0[0], agg0[1], deg[0], deg[1],
                 W_self0, W_neigh0, b0.reshape(1, H))
  agg1, = _sc_agg(h1, src, dst, z_d)
  score, pls = _tc_final(h1, agg1[0], agg1[1], deg[0], deg[1],
                         W_self1, W_neigh1, b1.reshape(1, H),
                         W_pred, b_pred.reshape(1, C))
  return (score, pls)


# speedup vs baseline: 1.2263x; 1.2263x over previous
import functools

import jax
import jax.numpy as jnp
from jax import lax
from jax.experimental import pallas as pl
from jax.experimental.pallas import tpu as pltpu
from jax.experimental.pallas import tpu_sc as plsc

N = 10000
E = 320000
D = 128
H = 128
C = 40

NC = 2
NS = 16
NW = NC * NS
RPT = 632
N_PAD = RPT * NS
CHUNK = 80
NCHUNK = 125
EPT = CHUNK * NCHUNK
E_PAD = NW * EPT

BN = 1000
GRID = N // BN


def _make_sc_agg(with_deg: bool):
  mesh = plsc.VectorSubcoreMesh(core_axis_name="c", subcore_axis_name="s")
  out_type = [jax.ShapeDtypeStruct((NC, N_PAD, D), jnp.float32)]
  scratch = [
      pltpu.VMEM((CHUNK,), jnp.int32),
      pltpu.VMEM((CHUNK,), jnp.int32),
      pltpu.VMEM((CHUNK, D), jnp.float32),
      pltpu.VMEM_SHARED((N_PAD, D), jnp.float32),
      pltpu.SemaphoreType.DMA,
  ]
  if with_deg:
    out_type.append(jax.ShapeDtypeStruct((NC, N_PAD, D), jnp.float32))
    scratch.append(pltpu.VMEM((CHUNK, D), jnp.float32))

  def body(*refs):
    if with_deg:
      (h_hbm, src_hbm, dst_hbm, z_d, ones_hbm, out_agg, out_deg,
       src_v, dst_v, rows_v, agg_sh, sem, ones_v) = refs
    else:
      (h_hbm, src_hbm, dst_hbm, z_d, out_agg,
       src_v, dst_v, rows_v, agg_sh, sem) = refs
    cid = lax.axis_index("c")
    sid = lax.axis_index("s")
    base = (cid * NS + sid) * EPT
    my_rows = pl.ds(sid * RPT, RPT)

    pltpu.sync_copy(z_d, agg_sh.at[my_rows])
    if with_deg:
      pltpu.sync_copy(ones_hbm, ones_v)
    plsc.subcore_barrier()

    if with_deg:
      def dstep(i, carry):
        off = base + i * CHUNK
        pltpu.sync_copy(dst_hbm.at[pl.ds(off, CHUNK)], dst_v)
        pltpu.sync_copy(ones_v, agg_sh.at[dst_v], add=True)
        return carry

      lax.fori_loop(0, NCHUNK, dstep, 0)
      plsc.subcore_barrier()
      pltpu.sync_copy(agg_sh.at[my_rows], out_deg.at[cid, my_rows])
      pltpu.sync_copy(z_d, agg_sh.at[my_rows])
      plsc.subcore_barrier()

    def step(i, carry):
      off = base + i * CHUNK
      pltpu.sync_copy(src_hbm.at[pl.ds(off, CHUNK)], src_v)
      pltpu.sync_copy(dst_hbm.at[pl.ds(off, CHUNK)], dst_v)
      pltpu.async_copy(h_hbm.at[src_v], rows_v, sem).wait()
      pltpu.sync_copy(rows_v, agg_sh.at[dst_v], add=True)
      return carry

    lax.fori_loop(0, NCHUNK, step, 0)
    plsc.subcore_barrier()
    pltpu.sync_copy(agg_sh.at[my_rows], out_agg.at[cid, my_rows])

  return pl.kernel(body, out_type=out_type, mesh=mesh, scratch_types=scratch)


_sc_agg_deg = _make_sc_agg(True)
_sc_agg = _make_sc_agg(False)


def _tc_layer_body(h_ref, aggA_ref, aggB_ref, degA_ref, degB_ref,
                   ws_ref, wn_ref, b_ref, out_ref):
  deg = degA_ref[:, :1] + degB_ref[:, :1]
  inv = 1.0 / jnp.maximum(deg, 1.0)
  agg = (aggA_ref[...] + aggB_ref[...]) * inv
  acc = jnp.dot(h_ref[...], ws_ref[...], preferred_element_type=jnp.float32)
  acc += jnp.dot(agg, wn_ref[...], preferred_element_type=jnp.float32)
  out_ref[...] = jnp.maximum(acc + b_ref[...], 0.0)


def _tc_final_body(h_ref, aggA_ref, aggB_ref, degA_ref, degB_ref,
                   ws_ref, wn_ref, b_ref, wp_ref, bp_ref,
                   score_ref, pls_ref):
  i = pl.program_id(0)
  deg = degA_ref[:, :1] + degB_ref[:, :1]
  inv = 1.0 / jnp.maximum(deg, 1.0)
  agg = (aggA_ref[...] + aggB_ref[...]) * inv
  acc = jnp.dot(h_ref[...], ws_ref[...], preferred_element_type=jnp.float32)
  acc += jnp.dot(agg, wn_ref[...], preferred_element_type=jnp.float32)
  h2 = jnp.maximum(acc + b_ref[...], 0.0)

  @pl.when(i == 0)
  def _():
    pls_ref[...] = jnp.zeros_like(pls_ref)

  pls_ref[...] += jnp.sum(h2, axis=0, keepdims=True)

  @pl.when(i == pl.num_programs(0) - 1)
  def _():
    pls = pls_ref[...] * (1.0 / N)
    pls_ref[...] = pls
    score_ref[...] = (
        jnp.dot(pls, wp_ref[...], preferred_element_type=jnp.float32)
        + bp_ref[...])


def _row_blocks(n_cols):
  return pl.BlockSpec((BN, n_cols), lambda i: (i, 0))


def _full(shape):
  return pl.BlockSpec(shape, lambda i: tuple(0 for _ in shape))


_tc_layer = pl.pallas_call(
    _tc_layer_body,
    out_shape=jax.ShapeDtypeStruct((N, H), jnp.float32),
    grid=(GRID,),
    in_specs=[
        _row_blocks(D), _row_blocks(D), _row_blocks(D),
        _row_blocks(D), _row_blocks(D),
        _full((D, H)), _full((D, H)), _full((1, H)),
    ],
    out_specs=_row_blocks(H),
)

_tc_final = pl.pallas_call(
    _tc_final_body,
    out_shape=(
        jax.ShapeDtypeStruct((1, C), jnp.float32),
        jax.ShapeDtypeStruct((1, H), jnp.float32),
    ),
    grid=(GRID,),
    in_specs=[
        _row_blocks(H), _row_blocks(H), _row_blocks(H),
        _row_blocks(D), _row_blocks(D),
        _full((H, H)), _full((H, H)), _full((1, H)),
        _full((H, C)), _full((1, C)),
    ],
    out_specs=(_full((1, C)), _full((1, H))),
)


def kernel(inputs, edge_index, W_self0, W_neigh0, b0, W_self1, W_neigh1, b1,
           W_pred, b_pred):
  src = edge_index[0]
  dst = edge_index[1]
  z_d = jnp.zeros((RPT, D), jnp.float32)
  ones = jnp.ones((CHUNK, D), jnp.float32)

  agg0, deg = _sc_agg_deg(inputs, src, dst, z_d, ones)
  h1 = _tc_layer(inputs, agg0[0], agg0[1], deg[0], deg[1],
                 W_self0, W_neigh0, b0.reshape(1, H))
  agg1, = _sc_agg(h1, src, dst, z_d)
  score, pls = _tc_final(h1, agg1[0], agg1[1], deg[0], deg[1],
                         W_self1, W_neigh1, b1.reshape(1, H),
                         W_pred, b_pred.reshape(1, C))
  return (score, pls)


# speedup vs baseline: 1.2268x; 1.0004x over previous
import functools

import jax
import jax.numpy as jnp
from jax import lax
from jax.experimental import pallas as pl
from jax.experimental.pallas import tpu as pltpu
from jax.experimental.pallas import tpu_sc as plsc

N = 10000
E = 320000
D = 128
H = 128
C = 40

NC = 2
NS = 16
NW = NC * NS
RPT = 632
N_PAD = RPT * NS
CHUNK = 80
NCHUNK = 125
EPT = CHUNK * NCHUNK
E_PAD = NW * EPT

BN = 1000
GRID = N // BN


def _make_sc_agg(with_deg: bool):
  mesh = plsc.VectorSubcoreMesh(core_axis_name="c", subcore_axis_name="s")
  out_type = [jax.ShapeDtypeStruct((NC, N_PAD, D), jnp.float32)]
  scratch = [
      pltpu.VMEM_SHARED((N_PAD, D), jnp.float32),
      pltpu.VMEM((CHUNK,), jnp.int32),
      pltpu.VMEM((CHUNK,), jnp.int32),
      pltpu.VMEM((CHUNK, D), jnp.float32),
      pltpu.SemaphoreType.DMA,
  ]
  if with_deg:
    out_type.append(jax.ShapeDtypeStruct((NC, N_PAD, D), jnp.float32))
    scratch.append(pltpu.VMEM((CHUNK, D), jnp.float32))

  def body(*refs):
    if with_deg:
      (h_hbm, src_hbm, dst_hbm, z_d, ones_hbm, out_agg, out_deg,
       agg_sh, src_v, dst_v, rows_v, sem, ones_v) = refs
    else:
      (h_hbm, src_hbm, dst_hbm, z_d, out_agg,
       agg_sh, src_v, dst_v, rows_v, sem) = refs
    cid = lax.axis_index("c")
    sid = lax.axis_index("s")
    base = (cid * NS + sid) * EPT
    my_rows = pl.ds(sid * RPT, RPT)

    pltpu.sync_copy(z_d, agg_sh.at[my_rows])
    if with_deg:
      pltpu.sync_copy(ones_hbm, ones_v)
    plsc.subcore_barrier()

    if with_deg:
      def dstep(i, carry):
        off = base + i * CHUNK
        pltpu.sync_copy(dst_hbm.at[pl.ds(off, CHUNK)], dst_v)
        pltpu.sync_copy(ones_v, agg_sh.at[dst_v], add=True)
        return carry

      lax.fori_loop(0, NCHUNK, dstep, 0)
      plsc.subcore_barrier()
      pltpu.sync_copy(agg_sh.at[my_rows], out_deg.at[cid, my_rows])
      pltpu.sync_copy(z_d, agg_sh.at[my_rows])
      plsc.subcore_barrier()

    def step(i, carry):
      off = base + i * CHUNK
      pltpu.sync_copy(src_hbm.at[pl.ds(off, CHUNK)], src_v)
      pltpu.sync_copy(dst_hbm.at[pl.ds(off, CHUNK)], dst_v)
      pltpu.async_copy(h_hbm.at[src_v], rows_v, sem).wait()
      pltpu.sync_copy(rows_v, agg_sh.at[dst_v], add=True)
      return carry

    lax.fori_loop(0, NCHUNK, step, 0)
    plsc.subcore_barrier()
    pltpu.sync_copy(agg_sh.at[my_rows], out_agg.at[cid, my_rows])

  return pl.kernel(body, out_type=out_type, mesh=mesh, scratch_types=scratch)


_sc_agg_deg = _make_sc_agg(True)
_sc_agg = _make_sc_agg(False)


def _tc_layer_body(h_ref, aggA_ref, aggB_ref, degA_ref, degB_ref,
                   ws_ref, wn_ref, b_ref, out_ref):
  deg = degA_ref[:, :1] + degB_ref[:, :1]
  inv = 1.0 / jnp.maximum(deg, 1.0)
  agg = (aggA_ref[...] + aggB_ref[...]) * inv
  acc = jnp.dot(h_ref[...], ws_ref[...], preferred_element_type=jnp.float32)
  acc += jnp.dot(agg, wn_ref[...], preferred_element_type=jnp.float32)
  out_ref[...] = jnp.maximum(acc + b_ref[...], 0.0)


def _tc_final_body(h_ref, aggA_ref, aggB_ref, degA_ref, degB_ref,
                   ws_ref, wn_ref, b_ref, wp_ref, bp_ref,
                   score_ref, pls_ref):
  i = pl.program_id(0)
  deg = degA_ref[:, :1] + degB_ref[:, :1]
  inv = 1.0 / jnp.maximum(deg, 1.0)
  agg = (aggA_ref[...] + aggB_ref[...]) * inv
  acc = jnp.dot(h_ref[...], ws_ref[...], preferred_element_type=jnp.float32)
  acc += jnp.dot(agg, wn_ref[...], preferred_element_type=jnp.float32)
  h2 = jnp.maximum(acc + b_ref[...], 0.0)

  @pl.when(i == 0)
  def _():
    pls_ref[...] = jnp.zeros_like(pls_ref)

  pls_ref[...] += jnp.sum(h2, axis=0, keepdims=True)

  @pl.when(i == pl.num_programs(0) - 1)
  def _():
    pls = pls_ref[...] * (1.0 / N)
    pls_ref[...] = pls
    score_ref[...] = (
        jnp.dot(pls, wp_ref[...], preferred_element_type=jnp.float32)
        + bp_ref[...])


def _row_blocks(n_cols):
  return pl.BlockSpec((BN, n_cols), lambda i: (i, 0))


def _full(shape):
  return pl.BlockSpec(shape, lambda i: tuple(0 for _ in shape))


_tc_layer = pl.pallas_call(
    _tc_layer_body,
    out_shape=jax.ShapeDtypeStruct((N, H), jnp.float32),
    grid=(GRID,),
    in_specs=[
        _row_blocks(D), _row_blocks(D), _row_blocks(D),
        _row_blocks(D), _row_blocks(D),
        _full((D, H)), _full((D, H)), _full((1, H)),
    ],
    out_specs=_row_blocks(H),
)

_tc_final = pl.pallas_call(
    _tc_final_body,
    out_shape=(
        jax.ShapeDtypeStruct((1, C), jnp.float32),
        jax.ShapeDtypeStruct((1, H), jnp.float32),
    ),
    grid=(GRID,),
    in_specs=[
        _row_blocks(H), _row_blocks(H), _row_blocks(H),
        _row_blocks(D), _row_blocks(D),
        _full((H, H)), _full((H, H)), _full((1, H)),
        _full((H, C)), _full((1, C)),
    ],
    out_specs=(_full((1, C)), _full((1, H))),
)


def kernel(inputs, edge_index, W_self0, W_neigh0, b0, W_self1, W_neigh1, b1,
           W_pred, b_pred):
  src = edge_index[0]
  dst = edge_index[1]
  z_d = jnp.zeros((RPT, D), jnp.float32)
  ones = jnp.ones((CHUNK, D), jnp.float32)

  agg0, deg = _sc_agg_deg(inputs, src, dst, z_d, ones)
  h1 = _tc_layer(inputs, agg0[0], agg0[1], deg[0], deg[1],
                 W_self0, W_neigh0, b0.reshape(1, H))
  agg1, = _sc_agg(h1, src, dst, z_d)
  score, pls = _tc_final(h1, agg1[0], agg1[1], deg[0], deg[1],
                         W_self1, W_neigh1, b1.reshape(1, H),
                         W_pred, b_pred.reshape(1, C))
  return (score, pls)
